# baseline (speedup 1.0000x reference)
import functools

import jax
import jax.numpy as jnp
from jax import lax
from jax.experimental import pallas as pl
from jax.experimental.pallas import tpu as pltpu
from jax.experimental.pallas import tpu_sc as plsc

B = 1024
D = 16
V = 100000
G = 8 * D

_NC = 2
_NS = 16
_NW = _NC * _NS
_B_PER_W = B // _NW


def _gather_groups(table, idx):
    mesh = plsc.VectorSubcoreMesh(core_axis_name="c", subcore_axis_name="s")

    @functools.partial(
        pl.kernel,
        mesh=mesh,
        out_type=jax.ShapeDtypeStruct((B, G), table.dtype),
        scratch_types=[
            pltpu.VMEM((_B_PER_W,), jnp.int32),
            pltpu.VMEM((_B_PER_W, G), table.dtype),
            pltpu.SemaphoreType.DMA,
        ],
    )
    def k(table_hbm, idx_hbm, out_hbm, idx_v, rows_v, sem):
        wid = lax.axis_index("s") * _NC + lax.axis_index("c")
        base = wid * _B_PER_W
        pltpu.sync_copy(idx_hbm.at[pl.ds(base, _B_PER_W)], idx_v)
        pltpu.async_copy(table_hbm.at[idx_v], rows_v, sem).wait()
        pltpu.sync_copy(rows_v, out_hbm.at[pl.ds(base, _B_PER_W)])

    return k(table, idx)


def _select_kernel(g_ref, r_ref, emb_ref):
    grp = g_ref[...]
    r = r_ref[...]
    emb = jnp.zeros((B, D), jnp.float32)
    for j in range(8):
        emb = emb + jnp.where(r == j, grp[:, j * D:(j + 1) * D], 0.0)
    emb_ref[...] = emb.astype(jnp.bfloat16)


def _select(groups, r):
    return pl.pallas_call(
        _select_kernel,
        out_shape=jax.ShapeDtypeStruct((B, D), jnp.bfloat16),
    )(groups, r)


_VB = 2048


def _matmul_kernel(emb_ref, w_ref, out_ref):
    out_ref[...] = lax.dot_general(
        emb_ref[...],
        w_ref[...].astype(jnp.bfloat16),
        dimension_numbers=(((1,), (1,)), ((), ())),
        preferred_element_type=jnp.float32,
    )


def _logits(emb_bf16, w_out):
    grid = (V + _VB - 1) // _VB
    return pl.pallas_call(
        _matmul_kernel,
        grid=(grid,),
        in_specs=[
            pl.BlockSpec((B, D), lambda i: (0, 0)),
            pl.BlockSpec((_VB, D), lambda i: (i, 0)),
        ],
        out_specs=pl.BlockSpec((B, _VB), lambda i: (0, i)),
        out_shape=jax.ShapeDtypeStruct((B, V), jnp.float32),
        compiler_params=pltpu.CompilerParams(
            dimension_semantics=("parallel",),
        ),
    )(emb_bf16, w_out)


def kernel(x, W_in, W_out):
    idx = x.astype(jnp.int32)
    table = W_in.reshape(V // 8, G)
    groups = _gather_groups(table, idx >> 3)
    emb = _select(groups, (idx & 7).reshape(B, 1))
    return _logits(emb, W_out)


# speedup vs baseline: 3.0451x; 3.0451x over previous
import functools

import jax
import jax.numpy as jnp
from jax import lax
from jax.experimental import pallas as pl
from jax.experimental.pallas import tpu as pltpu
from jax.experimental.pallas import tpu_sc as plsc

B = 1024
D = 16
V = 100000
G = 8 * D

_NC = 2
_NS = 16
_NW = _NC * _NS
_B_PER_W = B // _NW


def _gather_groups(table, idx):
    mesh = plsc.VectorSubcoreMesh(core_axis_name="c", subcore_axis_name="s")

    @functools.partial(
        pl.kernel,
        mesh=mesh,
        out_type=jax.ShapeDtypeStruct((B, G), table.dtype),
        scratch_types=[
            pltpu.VMEM((_B_PER_W,), jnp.int32),
            pltpu.VMEM((_B_PER_W, G), table.dtype),
            pltpu.SemaphoreType.DMA,
        ],
    )
    def k(table_hbm, idx_hbm, out_hbm, idx_v, rows_v, sem):
        wid = lax.axis_index("s") * _NC + lax.axis_index("c")
        base = wid * _B_PER_W
        pltpu.sync_copy(idx_hbm.at[pl.ds(base, _B_PER_W)], idx_v)
        pltpu.async_copy(table_hbm.at[idx_v], rows_v, sem).wait()
        pltpu.sync_copy(rows_v, out_hbm.at[pl.ds(base, _B_PER_W)])

    return k(table, idx)


def _select_kernel(g_ref, r_ref, emb_ref):
    grp = g_ref[...]
    r = r_ref[...]
    emb = jnp.zeros((B, D), jnp.float32)
    for j in range(8):
        emb = emb + jnp.where(r == j, grp[:, j * D:(j + 1) * D], 0.0)
    emb_ref[...] = emb.astype(jnp.bfloat16)


def _select(groups, r):
    return pl.pallas_call(
        _select_kernel,
        out_shape=jax.ShapeDtypeStruct((B, D), jnp.bfloat16),
    )(groups, r)


_VB = 2048


def _matmul_kernel(w_ref, emb_ref, out_ref):
    out_ref[...] = lax.dot_general(
        w_ref[...].astype(jnp.bfloat16),
        emb_ref[...],
        dimension_numbers=(((0,), (1,)), ((), ())),
        preferred_element_type=jnp.float32,
    )


def _logits_t(w_out_t, emb_bf16):
    grid = (V + _VB - 1) // _VB
    return pl.pallas_call(
        _matmul_kernel,
        grid=(grid,),
        in_specs=[
            pl.BlockSpec((D, _VB), lambda i: (0, i)),
            pl.BlockSpec((B, D), lambda i: (0, 0)),
        ],
        out_specs=pl.BlockSpec((_VB, B), lambda i: (i, 0)),
        out_shape=jax.ShapeDtypeStruct((V, B), jnp.float32),
        compiler_params=pltpu.CompilerParams(
            dimension_semantics=("parallel",),
        ),
    )(w_out_t, emb_bf16)


def kernel(x, W_in, W_out):
    idx = x.astype(jnp.int32)
    table = W_in.reshape(V // 8, G)
    groups = _gather_groups(table, idx >> 3)
    emb = _select(groups, (idx & 7).reshape(B, 1))
    return _logits_t(W_out.T, emb).T


# speedup vs baseline: 3.2569x; 1.0696x over previous
import functools

import jax
import jax.numpy as jnp
from jax import lax
from jax.experimental import pallas as pl
from jax.experimental.pallas import tpu as pltpu
from jax.experimental.pallas import tpu_sc as plsc

B = 1024
D = 16
V = 100000
G = 8 * D

_NC = 2
_NS = 16
_NW = _NC * _NS
_B_PER_W = B // _NW


def _gather_groups(table, idx):
    mesh = plsc.VectorSubcoreMesh(core_axis_name="c", subcore_axis_name="s")

    @functools.partial(
        pl.kernel,
        mesh=mesh,
        out_type=jax.ShapeDtypeStruct((B, G), table.dtype),
        scratch_types=[
            pltpu.VMEM((_B_PER_W,), jnp.int32),
            pltpu.VMEM((_B_PER_W, G), table.dtype),
            pltpu.SemaphoreType.DMA,
        ],
    )
    def k(table_hbm, idx_hbm, out_hbm, idx_v, rows_v, sem):
        wid = lax.axis_index("s") * _NC + lax.axis_index("c")
        base = wid * _B_PER_W
        pltpu.sync_copy(idx_hbm.at[pl.ds(base, _B_PER_W)], idx_v)
        pltpu.async_copy(table_hbm.at[idx_v], rows_v, sem).wait()
        pltpu.sync_copy(rows_v, out_hbm.at[pl.ds(base, _B_PER_W)])

    return k(table, idx)


_LB = 8192


def _regroup_kernel(in_ref, out_ref):
    x = in_ref[...]
    w = x.T.reshape(_LB // 8, 8, D)
    out_ref[...] = jnp.concatenate([w[:, j, :] for j in range(8)], axis=1)


def _regroup(w_in_t):
    grid = (V + _LB - 1) // _LB
    return pl.pallas_call(
        _regroup_kernel,
        grid=(grid,),
        in_specs=[pl.BlockSpec((D, _LB), lambda i: (0, i))],
        out_specs=pl.BlockSpec((_LB // 8, G), lambda i: (i, 0)),
        out_shape=jax.ShapeDtypeStruct((grid * _LB // 8, G), jnp.float32),
        compiler_params=pltpu.CompilerParams(
            dimension_semantics=("parallel",),
        ),
    )(w_in_t)


def _select_kernel(g_ref, r_ref, emb_ref):
    grp = g_ref[...]
    r = r_ref[...]
    emb = jnp.zeros((B, D), jnp.float32)
    for j in range(8):
        emb = emb + jnp.where(r == j, grp[:, j * D:(j + 1) * D], 0.0)
    emb_ref[...] = emb.astype(jnp.bfloat16)


def _select(groups, r):
    return pl.pallas_call(
        _select_kernel,
        out_shape=jax.ShapeDtypeStruct((B, D), jnp.bfloat16),
    )(groups, r)


_VB = 2048


def _matmul_kernel(w_ref, emb_ref, out_ref):
    out_ref[...] = lax.dot_general(
        w_ref[...].astype(jnp.bfloat16),
        emb_ref[...],
        dimension_numbers=(((0,), (1,)), ((), ())),
        preferred_element_type=jnp.float32,
    )


def _logits_t(w_out_t, emb_bf16):
    grid = (V + _VB - 1) // _VB
    return pl.pallas_call(
        _matmul_kernel,
        grid=(grid,),
        in_specs=[
            pl.BlockSpec((D, _VB), lambda i: (0, i)),
            pl.BlockSpec((B, D), lambda i: (0, 0)),
        ],
        out_specs=pl.BlockSpec((_VB, B), lambda i: (i, 0)),
        out_shape=jax.ShapeDtypeStruct((V, B), jnp.float32),
        compiler_params=pltpu.CompilerParams(
            dimension_semantics=("parallel",),
        ),
    )(w_out_t, emb_bf16)


def kernel(x, W_in, W_out):
    idx = x.astype(jnp.int32)
    table = _regroup(W_in.T)
    groups = _gather_groups(table, idx >> 3)
    emb = _select(groups, (idx & 7).reshape(B, 1))
    return _logits_t(W_out.T, emb).T
